# baseline (speedup 1.0000x reference)
import functools

import jax
import jax.numpy as jnp
from jax import lax
from jax.experimental import pallas as pl
from jax.experimental.pallas import tpu as pltpu
from jax.experimental.pallas import tpu_sc as plsc

_N = 10000
_E = 40000
_NSUB = 16
_CHUNK = 128
_CPS = 20
_EPAD = _NSUB * _CPS * _CHUNK
_HALF = 4992
_ACCR = 5120
_RPS = 320
_C0R = 312
_OUTR = _HALF + _ACCR
_DUMMY = _ACCR - 1


def _seg_scatter():
    mesh = plsc.VectorSubcoreMesh(core_axis_name="c", subcore_axis_name="s")

    @functools.partial(
        pl.kernel,
        out_type=jax.ShapeDtypeStruct((_OUTR, 128), jnp.float32),
        mesh=mesh,
        scratch_types=[
            pltpu.VMEM((_CPS, _CHUNK), jnp.int32),
            pltpu.VMEM((_CPS, _CHUNK), jnp.int32),
            pltpu.VMEM((2, _CHUNK, 128), jnp.float32),
            pltpu.VMEM((_RPS, 128), jnp.float32),
            pltpu.VMEM_SHARED((_ACCR, 128), jnp.float32),
            pltpu.SemaphoreType.DMA,
        ],
        compiler_params=pltpu.CompilerParams(needs_layout_passes=False),
    )
    def scatter_kernel(ytab, srcr, dstn, zeros, out, src_v, dst_v, rows_v,
                       bounce_v, acc, sem):
        c = lax.axis_index("c")
        s = lax.axis_index("s")
        pltpu.sync_copy(zeros, bounce_v)
        pltpu.sync_copy(bounce_v, acc.at[pl.ds(s * _RPS, _RPS)])
        pltpu.sync_copy(srcr.at[c, s], src_v)
        pltpu.sync_copy(dstn.at[c, s], dst_v)
        plsc.subcore_barrier()

        def count(j, n):
            f = dst_v[j, pl.ds(0, 16)]
            return n + jnp.where(jnp.min(f) != _DUMMY, 1, 0)

        cnt = lax.fori_loop(0, _CPS, count, 0)

        def chunk(j, carry):
            pltpu.make_async_copy(ytab.at[src_v.at[j]], rows_v.at[j % 2], sem).wait()

            @pl.when(j + 1 < cnt)
            def _():
                pltpu.async_copy(ytab.at[src_v.at[j + 1]], rows_v.at[(j + 1) % 2],
                                 sem)

            pltpu.sync_copy(rows_v.at[j % 2], acc.at[dst_v.at[j]], add=True)
            return carry

        @pl.when(cnt > 0)
        def _():
            pltpu.async_copy(ytab.at[src_v.at[0]], rows_v.at[0], sem)

        lax.fori_loop(0, cnt, chunk, 0)
        plsc.subcore_barrier()

        @pl.when(c == 0)
        def _():
            pltpu.sync_copy(acc.at[pl.ds(s * _C0R, _C0R)],
                            bounce_v.at[pl.ds(0, _C0R)])
            pltpu.sync_copy(bounce_v.at[pl.ds(0, _C0R)],
                            out.at[pl.ds(s * _C0R, _C0R)])

        @pl.when(c == 1)
        def _():
            pltpu.sync_copy(acc.at[pl.ds(s * _RPS, _RPS)], bounce_v)
            pltpu.sync_copy(bounce_v, out.at[pl.ds(_HALF + s * _RPS, _RPS)])

    return scatter_kernel


def _proj_first(x, wt, b, bn):
    n, din = x.shape

    def body(x_ref, wt_ref, b_ref, ya_ref, yb_ref, s_ref):
        z = jnp.dot(x_ref[...], wt_ref[...], preferred_element_type=jnp.float32)
        ya_ref[...] = z[:, :128]
        yb_ref[...] = z[:, 128:256]
        s_ref[...] = z[:, 256:] + b_ref[...]

    return pl.pallas_call(
        body,
        grid=(n // bn,),
        in_specs=[
            pl.BlockSpec((bn, din), lambda i: (i, 0)),
            pl.BlockSpec(wt.shape, lambda i: (0, 0)),
            pl.BlockSpec((1, 256), lambda i: (0, 0)),
        ],
        out_specs=[
            pl.BlockSpec((bn, 128), lambda i: (i, 0)),
            pl.BlockSpec((bn, 128), lambda i: (i, 0)),
            pl.BlockSpec((bn, 256), lambda i: (i, 0)),
        ],
        out_shape=[
            jax.ShapeDtypeStruct((n, 128), jnp.float32),
            jax.ShapeDtypeStruct((n, 128), jnp.float32),
            jax.ShapeDtypeStruct((n, 256), jnp.float32),
        ],
    )(x, wt, b)


def _proj_l2(a0, a1, s, wt, b, bn):
    n = s.shape[0]

    def body(a0_ref, a1_ref, s_ref, wt_ref, b_ref, y_ref, s2_ref):
        h0 = jnp.maximum(a0_ref[...] + s_ref[:, :128], 0.0)
        h1 = jnp.maximum(a1_ref[...] + s_ref[:, 128:], 0.0)
        z = (jnp.dot(h0, wt_ref[:128], preferred_element_type=jnp.float32)
             + jnp.dot(h1, wt_ref[128:], preferred_element_type=jnp.float32))
        y_ref[...] = z[:, :128]
        s2_ref[...] = z[:, 128:] + b_ref[...]

    return pl.pallas_call(
        body,
        grid=(n // bn,),
        in_specs=[
            pl.BlockSpec((bn, 128), lambda i: (i, 0)),
            pl.BlockSpec((bn, 128), lambda i: (i, 0)),
            pl.BlockSpec((bn, 256), lambda i: (i, 0)),
            pl.BlockSpec(wt.shape, lambda i: (0, 0)),
            pl.BlockSpec((1, 128), lambda i: (0, 0)),
        ],
        out_specs=[
            pl.BlockSpec((bn, 128), lambda i: (i, 0)),
            pl.BlockSpec((bn, 128), lambda i: (i, 0)),
        ],
        out_shape=[
            jax.ShapeDtypeStruct((n, 128), jnp.float32),
            jax.ShapeDtypeStruct((n, 128), jnp.float32),
        ],
    )(a0, a1, s, wt, b)


def _proj_l3(a2, s, wt, b, bn):
    n = s.shape[0]

    def body(a_ref, s_ref, wt_ref, b_ref, y_ref, s3_ref):
        h = jnp.maximum(a_ref[...] + s_ref[...], 0.0)
        z = jnp.dot(h, wt_ref[...], preferred_element_type=jnp.float32)
        y_ref[...] = jnp.concatenate([z[:, :64], jnp.zeros_like(z[:, :64])], axis=1)
        s3_ref[...] = z[:, 64:] + b_ref[...]

    return pl.pallas_call(
        body,
        grid=(n // bn,),
        in_specs=[
            pl.BlockSpec((bn, 128), lambda i: (i, 0)),
            pl.BlockSpec((bn, 128), lambda i: (i, 0)),
            pl.BlockSpec(wt.shape, lambda i: (0, 0)),
            pl.BlockSpec((1, 64), lambda i: (0, 0)),
        ],
        out_specs=[
            pl.BlockSpec((bn, 128), lambda i: (i, 0)),
            pl.BlockSpec((bn, 64), lambda i: (i, 0)),
        ],
        out_shape=[
            jax.ShapeDtypeStruct((n, 128), jnp.float32),
            jax.ShapeDtypeStruct((n, 64), jnp.float32),
        ],
    )(a2, s, wt, b)


def _proj_head(a3, s, wt, b, bn):
    n = s.shape[0]

    def body(a_ref, s_ref, wt_ref, b_ref, o_ref):
        h = jnp.maximum(a_ref[:, :64] + s_ref[...], 0.0)
        z = jnp.dot(h, wt_ref[...], preferred_element_type=jnp.float32) + b_ref[...]
        col = lax.broadcasted_iota(jnp.int32, z.shape, 1)
        m = (col >= 3) & (col < 7)
        nrm2 = jnp.sum(jnp.where(m, z * z, 0.0), axis=1, keepdims=True)
        denom = jnp.maximum(jnp.sqrt(nrm2), 1e-12)
        o_ref[...] = jnp.where(m, z / denom, z)

    return pl.pallas_call(
        body,
        grid=(n // bn,),
        in_specs=[
            pl.BlockSpec((bn, 128), lambda i: (i, 0)),
            pl.BlockSpec((bn, 64), lambda i: (i, 0)),
            pl.BlockSpec(wt.shape, lambda i: (0, 0)),
            pl.BlockSpec((1, 8), lambda i: (0, 0)),
        ],
        out_specs=pl.BlockSpec((bn, 8), lambda i: (i, 0)),
        out_shape=jax.ShapeDtypeStruct((n, 8), jnp.float32),
    )(a3, s, wt, b)


def kernel(x, edge_index, W1r, W1s, b1, W2r, W2s, b2, W3r, W3s, b3, Wp, bp, Wo, bo):
    src = edge_index[0]
    dst = edge_index[1]
    pad = _EPAD - _E
    srcp = jnp.concatenate([src, jnp.zeros((pad,), jnp.int32)])
    dstp = jnp.concatenate([dst, jnp.full((pad,), _N, jnp.int32)])
    half = dstp >= _HALF
    order0 = jnp.argsort(half.astype(jnp.int32), stable=True)
    order1 = order0[::-1]
    n0 = _EPAD - jnp.sum(half.astype(jnp.int32))
    pos = jnp.arange(_EPAD, dtype=jnp.int32)
    loc = jnp.where(half, jnp.minimum(dstp - _HALF, _DUMMY), dstp)

    def _lists(order, cntc):
        sl = jnp.where(pos < cntc, srcp[order], 0)
        dl = jnp.where(pos < cntc, loc[order], _DUMMY)
        sl = sl.reshape(_CPS * _CHUNK, _NSUB).T.reshape(_NSUB, _CPS, _CHUNK)
        dl = dl.reshape(_CPS * _CHUNK, _NSUB).T.reshape(_NSUB, _CPS, _CHUNK)
        return sl, dl

    s0, d0 = _lists(order0, n0)
    s1, d1 = _lists(order1, _EPAD - n0)
    srcr = jnp.stack([s0, s1])
    dstn = jnp.stack([d0, d1])


    wt1 = jnp.concatenate([W1r, W1s], axis=0).T
    wt2 = jnp.concatenate([W2r, W2s], axis=0).T
    wt3 = jnp.concatenate([W3r, W3s], axis=0).T
    wth = jnp.concatenate(
        [Wp, Wo, jnp.zeros((1, 64), jnp.float32)], axis=0).T
    bh = jnp.concatenate([bp, bo, jnp.zeros((1,), jnp.float32)]).reshape(1, 8)
    zer = jnp.zeros((_RPS, 128), jnp.float32)
    sc = _seg_scatter()

    y1a, y1b, s1 = _proj_first(x, wt1, b1.reshape(1, -1), 1000)
    a0 = sc(y1a, srcr, dstn, zer)
    a1 = sc(y1b, srcr, dstn, zer)
    y2, s2 = _proj_l2(a0, a1, s1, wt2, b2.reshape(1, -1), 1000)
    a2 = sc(y2, srcr, dstn, zer)
    y3, s3 = _proj_l3(a2, s2, wt3, b3.reshape(1, -1), 1000)
    a3 = sc(y3, srcr, dstn, zer)
    out8 = _proj_head(a3, s3, wth, bh, 1000)
    return (out8[:, 0:3], out8[:, 3:7])


# speedup vs baseline: 1.3575x; 1.3575x over previous
import functools

import jax
import jax.numpy as jnp
from jax import lax
from jax.experimental import pallas as pl
from jax.experimental.pallas import tpu as pltpu
from jax.experimental.pallas import tpu_sc as plsc

_N = 10000
_E = 40000
_NSUB = 16
_CHUNK = 128
_CPS = 20
_EPAD = _NSUB * _CPS * _CHUNK
_HALF = 4992
_ACCR = 5120
_RPS = 320
_C0R = 312
_OUTR = _HALF + _ACCR
_DUMMY = _ACCR - 1


def _seg_scatter(ntab):
    mesh = plsc.VectorSubcoreMesh(core_axis_name="c", subcore_axis_name="s")

    @functools.partial(
        pl.kernel,
        out_type=[jax.ShapeDtypeStruct((_OUTR, 128), jnp.float32)] * ntab,
        mesh=mesh,
        scratch_types=[
            pltpu.VMEM((_CPS, _CHUNK), jnp.int32),
            pltpu.VMEM((_CPS, _CHUNK), jnp.int32),
            pltpu.VMEM((2, _CHUNK, 128), jnp.float32),
            pltpu.VMEM((_RPS, 128), jnp.float32),
            pltpu.VMEM_SHARED((_ACCR, 128), jnp.float32),
            pltpu.SemaphoreType.DMA,
        ],
        compiler_params=pltpu.CompilerParams(needs_layout_passes=False),
    )
    def scatter_kernel(*refs):
        ytabs = refs[:ntab]
        srcr, dstn, zeros = refs[ntab:ntab + 3]
        outs = refs[ntab + 3:2 * ntab + 3]
        src_v, dst_v, rows_v, bounce_v, acc, sem = refs[2 * ntab + 3:]
        c = lax.axis_index("c")
        s = lax.axis_index("s")
        pltpu.sync_copy(srcr.at[s], src_v)
        pltpu.sync_copy(dstn.at[c, s], dst_v)

        for t in range(ntab):
            ytab = ytabs[t]
            out = outs[t]
            pltpu.sync_copy(zeros, bounce_v)
            pltpu.sync_copy(bounce_v, acc.at[pl.ds(s * _RPS, _RPS)])
            plsc.subcore_barrier()

            def chunk(j, carry, ytab=ytab):
                pltpu.make_async_copy(ytab.at[src_v.at[j]], rows_v.at[j % 2],
                                      sem).wait()

                @pl.when(j + 1 < _CPS)
                def _():
                    pltpu.async_copy(ytab.at[src_v.at[j + 1]],
                                     rows_v.at[(j + 1) % 2], sem)

                pltpu.sync_copy(rows_v.at[j % 2], acc.at[dst_v.at[j]], add=True)
                return carry

            pltpu.async_copy(ytab.at[src_v.at[0]], rows_v.at[0], sem)
            lax.fori_loop(0, _CPS, chunk, 0)
            plsc.subcore_barrier()

            @pl.when(c == 0)
            def _():
                pltpu.sync_copy(acc.at[pl.ds(s * _C0R, _C0R)],
                                bounce_v.at[pl.ds(0, _C0R)])
                pltpu.sync_copy(bounce_v.at[pl.ds(0, _C0R)],
                                out.at[pl.ds(s * _C0R, _C0R)])

            @pl.when(c == 1)
            def _():
                pltpu.sync_copy(acc.at[pl.ds(s * _RPS, _RPS)], bounce_v)
                pltpu.sync_copy(bounce_v, out.at[pl.ds(_HALF + s * _RPS, _RPS)])

            if t + 1 < ntab:
                plsc.subcore_barrier()

    return scatter_kernel


def _proj_first(x, wt, b, bn):
    n, din = x.shape

    def body(x_ref, wt_ref, b_ref, ya_ref, yb_ref, s_ref):
        z = jnp.dot(x_ref[...], wt_ref[...], preferred_element_type=jnp.float32)
        ya_ref[...] = z[:, :128]
        yb_ref[...] = z[:, 128:256]
        s_ref[...] = z[:, 256:] + b_ref[...]

    return pl.pallas_call(
        body,
        grid=(n // bn,),
        in_specs=[
            pl.BlockSpec((bn, din), lambda i: (i, 0)),
            pl.BlockSpec(wt.shape, lambda i: (0, 0)),
            pl.BlockSpec((1, 256), lambda i: (0, 0)),
        ],
        out_specs=[
            pl.BlockSpec((bn, 128), lambda i: (i, 0)),
            pl.BlockSpec((bn, 128), lambda i: (i, 0)),
            pl.BlockSpec((bn, 256), lambda i: (i, 0)),
        ],
        out_shape=[
            jax.ShapeDtypeStruct((n, 128), jnp.float32),
            jax.ShapeDtypeStruct((n, 128), jnp.float32),
            jax.ShapeDtypeStruct((n, 256), jnp.float32),
        ],
    )(x, wt, b)


def _proj_l2(a0, a1, s, wt, b, bn):
    n = s.shape[0]

    def body(a0_ref, a1_ref, s_ref, wt_ref, b_ref, y_ref, s2_ref):
        h0 = jnp.maximum(a0_ref[...] + s_ref[:, :128], 0.0)
        h1 = jnp.maximum(a1_ref[...] + s_ref[:, 128:], 0.0)
        z = (jnp.dot(h0, wt_ref[:128], preferred_element_type=jnp.float32)
             + jnp.dot(h1, wt_ref[128:], preferred_element_type=jnp.float32))
        y_ref[...] = z[:, :128]
        s2_ref[...] = z[:, 128:] + b_ref[...]

    return pl.pallas_call(
        body,
        grid=(n // bn,),
        in_specs=[
            pl.BlockSpec((bn, 128), lambda i: (i, 0)),
            pl.BlockSpec((bn, 128), lambda i: (i, 0)),
            pl.BlockSpec((bn, 256), lambda i: (i, 0)),
            pl.BlockSpec(wt.shape, lambda i: (0, 0)),
            pl.BlockSpec((1, 128), lambda i: (0, 0)),
        ],
        out_specs=[
            pl.BlockSpec((bn, 128), lambda i: (i, 0)),
            pl.BlockSpec((bn, 128), lambda i: (i, 0)),
        ],
        out_shape=[
            jax.ShapeDtypeStruct((n, 128), jnp.float32),
            jax.ShapeDtypeStruct((n, 128), jnp.float32),
        ],
    )(a0, a1, s, wt, b)


def _proj_l3(a2, s, wt, b, bn):
    n = s.shape[0]

    def body(a_ref, s_ref, wt_ref, b_ref, y_ref, s3_ref):
        h = jnp.maximum(a_ref[...] + s_ref[...], 0.0)
        z = jnp.dot(h, wt_ref[...], preferred_element_type=jnp.float32)
        y_ref[...] = jnp.concatenate([z[:, :64], jnp.zeros_like(z[:, :64])], axis=1)
        s3_ref[...] = z[:, 64:] + b_ref[...]

    return pl.pallas_call(
        body,
        grid=(n // bn,),
        in_specs=[
            pl.BlockSpec((bn, 128), lambda i: (i, 0)),
            pl.BlockSpec((bn, 128), lambda i: (i, 0)),
            pl.BlockSpec(wt.shape, lambda i: (0, 0)),
            pl.BlockSpec((1, 64), lambda i: (0, 0)),
        ],
        out_specs=[
            pl.BlockSpec((bn, 128), lambda i: (i, 0)),
            pl.BlockSpec((bn, 64), lambda i: (i, 0)),
        ],
        out_shape=[
            jax.ShapeDtypeStruct((n, 128), jnp.float32),
            jax.ShapeDtypeStruct((n, 64), jnp.float32),
        ],
    )(a2, s, wt, b)


def _proj_head(a3, s, wt, b, bn):
    n = s.shape[0]

    def body(a_ref, s_ref, wt_ref, b_ref, o_ref):
        h = jnp.maximum(a_ref[:, :64] + s_ref[...], 0.0)
        z = jnp.dot(h, wt_ref[...], preferred_element_type=jnp.float32) + b_ref[...]
        col = lax.broadcasted_iota(jnp.int32, z.shape, 1)
        m = (col >= 3) & (col < 7)
        nrm2 = jnp.sum(jnp.where(m, z * z, 0.0), axis=1, keepdims=True)
        denom = jnp.maximum(jnp.sqrt(nrm2), 1e-12)
        o_ref[...] = jnp.where(m, z / denom, z)

    return pl.pallas_call(
        body,
        grid=(n // bn,),
        in_specs=[
            pl.BlockSpec((bn, 128), lambda i: (i, 0)),
            pl.BlockSpec((bn, 64), lambda i: (i, 0)),
            pl.BlockSpec(wt.shape, lambda i: (0, 0)),
            pl.BlockSpec((1, 8), lambda i: (0, 0)),
        ],
        out_specs=pl.BlockSpec((bn, 8), lambda i: (i, 0)),
        out_shape=jax.ShapeDtypeStruct((n, 8), jnp.float32),
    )(a3, s, wt, b)


def kernel(x, edge_index, W1r, W1s, b1, W2r, W2s, b2, W3r, W3s, b3, Wp, bp, Wo, bo):
    src = edge_index[0]
    dst = edge_index[1]
    pad = _EPAD - _E
    srcr = jnp.concatenate(
        [src, jnp.zeros((pad,), jnp.int32)]).reshape(_NSUB, _CPS, _CHUNK)
    dstp = jnp.concatenate([dst, jnp.full((pad,), _N, jnp.int32)])
    pos = jnp.arange(_EPAD, dtype=jnp.int32)
    dum0 = _HALF + (pos % (_ACCR - _HALF))
    dum1 = (_N - _HALF + 1) + (pos % (_DUMMY - (_N - _HALF)))
    loc0 = jnp.where(dstp < _HALF, dstp, dum0).reshape(_NSUB, _CPS, _CHUNK)
    loc1 = jnp.where(dstp >= _HALF, jnp.minimum(dstp - _HALF, _DUMMY),
                     dum1).reshape(_NSUB, _CPS, _CHUNK)
    dstn = jnp.stack([loc0, loc1])


    wt1 = jnp.concatenate([W1r, W1s], axis=0).T
    wt2 = jnp.concatenate([W2r, W2s], axis=0).T
    wt3 = jnp.concatenate([W3r, W3s], axis=0).T
    wth = jnp.concatenate(
        [Wp, Wo, jnp.zeros((1, 64), jnp.float32)], axis=0).T
    bh = jnp.concatenate([bp, bo, jnp.zeros((1,), jnp.float32)]).reshape(1, 8)
    zer = jnp.zeros((_RPS, 128), jnp.float32)
    sc1 = _seg_scatter(1)
    sc2 = _seg_scatter(2)

    y1a, y1b, s1 = _proj_first(x, wt1, b1.reshape(1, -1), 1000)
    a0, a1 = sc2(y1a, y1b, srcr, dstn, zer)
    y2, s2 = _proj_l2(a0, a1, s1, wt2, b2.reshape(1, -1), 1000)
    a2, = sc1(y2, srcr, dstn, zer)
    y3, s3 = _proj_l3(a2, s2, wt3, b3.reshape(1, -1), 1000)
    a3, = sc1(y3, srcr, dstn, zer)
    out8 = _proj_head(a3, s3, wth, bh, 1000)
    return (out8[:, 0:3], out8[:, 3:7])


# speedup vs baseline: 1.4386x; 1.0598x over previous
import functools

import jax
import jax.numpy as jnp
from jax import lax
from jax.experimental import pallas as pl
from jax.experimental.pallas import tpu as pltpu
from jax.experimental.pallas import tpu_sc as plsc

_N = 10000
_E = 40000
_NSUB = 16
_CHUNK = 128
_CPS = 20
_EPAD = _NSUB * _CPS * _CHUNK
_HALF = 4992
_ACCR = 5120
_RPS = 320
_C0R = 312
_OUTR = _HALF + _ACCR
_DUMMY = _ACCR - 1


def _seg_scatter(ntab):
    mesh = plsc.VectorSubcoreMesh(core_axis_name="c", subcore_axis_name="s")

    @functools.partial(
        pl.kernel,
        out_type=[jax.ShapeDtypeStruct((_OUTR, 128), jnp.float32)] * ntab,
        mesh=mesh,
        scratch_types=[
            pltpu.VMEM((_CPS, _CHUNK), jnp.int32),
            pltpu.VMEM((_CPS, _CHUNK), jnp.int32),
            pltpu.VMEM((3, _CHUNK, 128), jnp.float32),
            pltpu.VMEM_SHARED((_ACCR, 128), jnp.float32),
            pltpu.SemaphoreType.DMA,
        ],
        compiler_params=pltpu.CompilerParams(needs_layout_passes=False),
    )
    def scatter_kernel(*refs):
        ytabs = refs[:ntab]
        srcr, dstn, zeros = refs[ntab:ntab + 3]
        outs = refs[ntab + 3:2 * ntab + 3]
        src_v, dst_v, rows_v, acc, sem = refs[2 * ntab + 3:]
        c = lax.axis_index("c")
        s = lax.axis_index("s")
        pltpu.sync_copy(srcr.at[s], src_v)
        pltpu.sync_copy(dstn.at[c, s], dst_v)

        for t in range(ntab):
            ytab = ytabs[t]
            out = outs[t]
            pltpu.sync_copy(zeros, rows_v.at[0])
            base = s * _RPS
            pltpu.sync_copy(rows_v.at[0], acc.at[pl.ds(base, _CHUNK)])
            pltpu.sync_copy(rows_v.at[0], acc.at[pl.ds(base + _CHUNK, _CHUNK)])
            pltpu.sync_copy(rows_v.at[0].at[pl.ds(0, _RPS - 2 * _CHUNK)],
                            acc.at[pl.ds(base + 2 * _CHUNK, _RPS - 2 * _CHUNK)])
            plsc.subcore_barrier()

            def chunk(j, carry, ytab=ytab):
                pltpu.make_async_copy(ytab.at[src_v.at[j]], rows_v.at[j % 3],
                                      sem).wait()

                @pl.when(j + 2 < _CPS)
                def _():
                    pltpu.async_copy(ytab.at[src_v.at[j + 2]],
                                     rows_v.at[(j + 2) % 3], sem)

                pltpu.sync_copy(rows_v.at[j % 3], acc.at[dst_v.at[j]], add=True)
                return carry

            for b in range(2):
                pltpu.async_copy(ytab.at[src_v.at[b]], rows_v.at[b], sem)
            lax.fori_loop(0, _CPS, chunk, 0)
            plsc.subcore_barrier()

            @pl.when(c == 0)
            def _():
                for p, (off, ln) in enumerate(((0, _CHUNK), (_CHUNK, _CHUNK),
                                               (2 * _CHUNK, _C0R - 2 * _CHUNK))):
                    piece = rows_v.at[p].at[pl.ds(0, ln)]
                    pltpu.sync_copy(acc.at[pl.ds(s * _C0R + off, ln)], piece)
                    pltpu.sync_copy(piece, out.at[pl.ds(s * _C0R + off, ln)])

            @pl.when(c == 1)
            def _():
                for p, (off, ln) in enumerate(((0, _CHUNK), (_CHUNK, _CHUNK),
                                               (2 * _CHUNK, _RPS - 2 * _CHUNK))):
                    piece = rows_v.at[p].at[pl.ds(0, ln)]
                    pltpu.sync_copy(acc.at[pl.ds(s * _RPS + off, ln)], piece)
                    pltpu.sync_copy(piece,
                                    out.at[pl.ds(_HALF + s * _RPS + off, ln)])

            if t + 1 < ntab:
                plsc.subcore_barrier()

    return scatter_kernel


def _proj_first(x, wt, b, bn):
    n, din = x.shape

    def body(x_ref, wt_ref, b_ref, ya_ref, yb_ref, s_ref):
        z = jnp.dot(x_ref[...], wt_ref[...], preferred_element_type=jnp.float32)
        ya_ref[...] = z[:, :128]
        yb_ref[...] = z[:, 128:256]
        s_ref[...] = z[:, 256:] + b_ref[...]

    return pl.pallas_call(
        body,
        grid=(n // bn,),
        in_specs=[
            pl.BlockSpec((bn, din), lambda i: (i, 0)),
            pl.BlockSpec(wt.shape, lambda i: (0, 0)),
            pl.BlockSpec((1, 256), lambda i: (0, 0)),
        ],
        out_specs=[
            pl.BlockSpec((bn, 128), lambda i: (i, 0)),
            pl.BlockSpec((bn, 128), lambda i: (i, 0)),
            pl.BlockSpec((bn, 256), lambda i: (i, 0)),
        ],
        out_shape=[
            jax.ShapeDtypeStruct((n, 128), jnp.float32),
            jax.ShapeDtypeStruct((n, 128), jnp.float32),
            jax.ShapeDtypeStruct((n, 256), jnp.float32),
        ],
    )(x, wt, b)


def _proj_l2(a0, a1, s, wt, b, bn):
    n = s.shape[0]

    def body(a0_ref, a1_ref, s_ref, wt_ref, b_ref, y_ref, s2_ref):
        h0 = jnp.maximum(a0_ref[...] + s_ref[:, :128], 0.0)
        h1 = jnp.maximum(a1_ref[...] + s_ref[:, 128:], 0.0)
        z = (jnp.dot(h0, wt_ref[:128], preferred_element_type=jnp.float32)
             + jnp.dot(h1, wt_ref[128:], preferred_element_type=jnp.float32))
        y_ref[...] = z[:, :128]
        s2_ref[...] = z[:, 128:] + b_ref[...]

    return pl.pallas_call(
        body,
        grid=(n // bn,),
        in_specs=[
            pl.BlockSpec((bn, 128), lambda i: (i, 0)),
            pl.BlockSpec((bn, 128), lambda i: (i, 0)),
            pl.BlockSpec((bn, 256), lambda i: (i, 0)),
            pl.BlockSpec(wt.shape, lambda i: (0, 0)),
            pl.BlockSpec((1, 128), lambda i: (0, 0)),
        ],
        out_specs=[
            pl.BlockSpec((bn, 128), lambda i: (i, 0)),
            pl.BlockSpec((bn, 128), lambda i: (i, 0)),
        ],
        out_shape=[
            jax.ShapeDtypeStruct((n, 128), jnp.float32),
            jax.ShapeDtypeStruct((n, 128), jnp.float32),
        ],
    )(a0, a1, s, wt, b)


def _proj_l3(a2, s, wt, b, bn):
    n = s.shape[0]

    def body(a_ref, s_ref, wt_ref, b_ref, y_ref, s3_ref):
        h = jnp.maximum(a_ref[...] + s_ref[...], 0.0)
        z = jnp.dot(h, wt_ref[...], preferred_element_type=jnp.float32)
        y_ref[...] = jnp.concatenate([z[:, :64], jnp.zeros_like(z[:, :64])], axis=1)
        s3_ref[...] = z[:, 64:] + b_ref[...]

    return pl.pallas_call(
        body,
        grid=(n // bn,),
        in_specs=[
            pl.BlockSpec((bn, 128), lambda i: (i, 0)),
            pl.BlockSpec((bn, 128), lambda i: (i, 0)),
            pl.BlockSpec(wt.shape, lambda i: (0, 0)),
            pl.BlockSpec((1, 64), lambda i: (0, 0)),
        ],
        out_specs=[
            pl.BlockSpec((bn, 128), lambda i: (i, 0)),
            pl.BlockSpec((bn, 64), lambda i: (i, 0)),
        ],
        out_shape=[
            jax.ShapeDtypeStruct((n, 128), jnp.float32),
            jax.ShapeDtypeStruct((n, 64), jnp.float32),
        ],
    )(a2, s, wt, b)


def _proj_head(a3, s, wt, b, bn):
    n = s.shape[0]

    def body(a_ref, s_ref, wt_ref, b_ref, o_ref):
        h = jnp.maximum(a_ref[:, :64] + s_ref[...], 0.0)
        z = jnp.dot(h, wt_ref[...], preferred_element_type=jnp.float32) + b_ref[...]
        col = lax.broadcasted_iota(jnp.int32, z.shape, 1)
        m = (col >= 3) & (col < 7)
        nrm2 = jnp.sum(jnp.where(m, z * z, 0.0), axis=1, keepdims=True)
        denom = jnp.maximum(jnp.sqrt(nrm2), 1e-12)
        o_ref[...] = jnp.where(m, z / denom, z)

    return pl.pallas_call(
        body,
        grid=(n // bn,),
        in_specs=[
            pl.BlockSpec((bn, 128), lambda i: (i, 0)),
            pl.BlockSpec((bn, 64), lambda i: (i, 0)),
            pl.BlockSpec(wt.shape, lambda i: (0, 0)),
            pl.BlockSpec((1, 8), lambda i: (0, 0)),
        ],
        out_specs=pl.BlockSpec((bn, 8), lambda i: (i, 0)),
        out_shape=jax.ShapeDtypeStruct((n, 8), jnp.float32),
    )(a3, s, wt, b)


def kernel(x, edge_index, W1r, W1s, b1, W2r, W2s, b2, W3r, W3s, b3, Wp, bp, Wo, bo):
    src = edge_index[0]
    dst = edge_index[1]
    pad = _EPAD - _E
    srcr = jnp.concatenate(
        [src, jnp.zeros((pad,), jnp.int32)]).reshape(_NSUB, _CPS, _CHUNK)
    dstp = jnp.concatenate([dst, jnp.full((pad,), _N, jnp.int32)])
    pos = jnp.arange(_EPAD, dtype=jnp.int32)
    dum0 = _HALF + (pos % (_ACCR - _HALF))
    dum1 = (_N - _HALF + 1) + (pos % (_DUMMY - (_N - _HALF)))
    loc0 = jnp.where(dstp < _HALF, dstp, dum0).reshape(_NSUB, _CPS, _CHUNK)
    loc1 = jnp.where(dstp >= _HALF, jnp.minimum(dstp - _HALF, _DUMMY),
                     dum1).reshape(_NSUB, _CPS, _CHUNK)
    dstn = jnp.stack([loc0, loc1])


    wt1 = jnp.concatenate([W1r, W1s], axis=0).T
    wt2 = jnp.concatenate([W2r, W2s], axis=0).T
    wt3 = jnp.concatenate([W3r, W3s], axis=0).T
    wth = jnp.concatenate(
        [Wp, Wo, jnp.zeros((1, 64), jnp.float32)], axis=0).T
    bh = jnp.concatenate([bp, bo, jnp.zeros((1,), jnp.float32)]).reshape(1, 8)
    zer = jnp.zeros((_CHUNK, 128), jnp.float32)
    sc1 = _seg_scatter(1)
    sc2 = _seg_scatter(2)

    y1a, y1b, s1 = _proj_first(x, wt1, b1.reshape(1, -1), 1000)
    a0, a1 = sc2(y1a, y1b, srcr, dstn, zer)
    y2, s2 = _proj_l2(a0, a1, s1, wt2, b2.reshape(1, -1), 1000)
    a2, = sc1(y2, srcr, dstn, zer)
    y3, s3 = _proj_l3(a2, s2, wt3, b3.reshape(1, -1), 1000)
    a3, = sc1(y3, srcr, dstn, zer)
    out8 = _proj_head(a3, s3, wth, bh, 1000)
    return (out8[:, 0:3], out8[:, 3:7])


# speedup vs baseline: 1.4394x; 1.0006x over previous
import functools

import jax
import jax.numpy as jnp
from jax import lax
from jax.experimental import pallas as pl
from jax.experimental.pallas import tpu as pltpu
from jax.experimental.pallas import tpu_sc as plsc

_N = 10000
_E = 40000
_NSUB = 16
_CHUNK = 128
_CPS = 20
_EPAD = _NSUB * _CPS * _CHUNK
_HALF = 4992
_ACCR = 5120
_RPS = 320
_C0R = 312
_OUTR = _HALF + _ACCR
_DUMMY = _ACCR - 1


def _seg_scatter(ntab):
    mesh = plsc.VectorSubcoreMesh(core_axis_name="c", subcore_axis_name="s")

    @functools.partial(
        pl.kernel,
        out_type=[jax.ShapeDtypeStruct((_OUTR, 128), jnp.float32)] * ntab,
        mesh=mesh,
        scratch_types=[
            pltpu.VMEM((_CPS, _CHUNK), jnp.int32),
            pltpu.VMEM((_CPS, _CHUNK), jnp.int32),
            pltpu.VMEM((3, _CHUNK, 128), jnp.float32),
            pltpu.VMEM_SHARED((_ACCR, 128), jnp.float32),
            pltpu.SemaphoreType.DMA,
        ],
        compiler_params=pltpu.CompilerParams(needs_layout_passes=False),
    )
    def scatter_kernel(*refs):
        ytabs = refs[:ntab]
        srcr, dstn, zeros = refs[ntab:ntab + 3]
        outs = refs[ntab + 3:2 * ntab + 3]
        src_v, dst_v, rows_v, acc, sem = refs[2 * ntab + 3:]
        c = lax.axis_index("c")
        s = lax.axis_index("s")
        pltpu.sync_copy(srcr.at[s], src_v)
        pltpu.sync_copy(dstn.at[c, s], dst_v)

        for t in range(ntab):
            ytab = ytabs[t]
            out = outs[t]
            pltpu.sync_copy(zeros, rows_v.at[0])
            base = s * _RPS
            pltpu.sync_copy(rows_v.at[0], acc.at[pl.ds(base, _CHUNK)])
            pltpu.sync_copy(rows_v.at[0], acc.at[pl.ds(base + _CHUNK, _CHUNK)])
            pltpu.sync_copy(rows_v.at[0].at[pl.ds(0, _RPS - 2 * _CHUNK)],
                            acc.at[pl.ds(base + 2 * _CHUNK, _RPS - 2 * _CHUNK)])
            plsc.subcore_barrier()

            def chunk(j, carry, ytab=ytab):
                pltpu.make_async_copy(ytab.at[src_v.at[j]], rows_v.at[j % 3],
                                      sem).wait()

                @pl.when(j + 2 < _CPS)
                def _():
                    pltpu.async_copy(ytab.at[src_v.at[j + 2]],
                                     rows_v.at[(j + 2) % 3], sem)

                pltpu.sync_copy(rows_v.at[j % 3], acc.at[dst_v.at[j]], add=True)
                return carry

            for b in range(2):
                pltpu.async_copy(ytab.at[src_v.at[b]], rows_v.at[b], sem)
            lax.fori_loop(0, _CPS, chunk, 0)
            plsc.subcore_barrier()

            @pl.when(c == 0)
            def _():
                for p, (off, ln) in enumerate(((0, _CHUNK), (_CHUNK, _CHUNK),
                                               (2 * _CHUNK, _C0R - 2 * _CHUNK))):
                    piece = rows_v.at[p].at[pl.ds(0, ln)]
                    pltpu.sync_copy(acc.at[pl.ds(s * _C0R + off, ln)], piece)
                    pltpu.sync_copy(piece, out.at[pl.ds(s * _C0R + off, ln)])

            @pl.when(c == 1)
            def _():
                for p, (off, ln) in enumerate(((0, _CHUNK), (_CHUNK, _CHUNK),
                                               (2 * _CHUNK, _RPS - 2 * _CHUNK))):
                    piece = rows_v.at[p].at[pl.ds(0, ln)]
                    pltpu.sync_copy(acc.at[pl.ds(s * _RPS + off, ln)], piece)
                    pltpu.sync_copy(piece,
                                    out.at[pl.ds(_HALF + s * _RPS + off, ln)])

            if t + 1 < ntab:
                plsc.subcore_barrier()

    return scatter_kernel


def _proj_first(x, wt, b, bn):
    n, din = x.shape

    def body(x_ref, wt_ref, b_ref, ya_ref, yb_ref, s_ref):
        z = jnp.dot(x_ref[...].astype(jnp.bfloat16),
                    wt_ref[...].astype(jnp.bfloat16),
                    preferred_element_type=jnp.float32)
        ya_ref[...] = z[:, :128]
        yb_ref[...] = z[:, 128:256]
        s_ref[...] = z[:, 256:] + b_ref[...]

    return pl.pallas_call(
        body,
        grid=(n // bn,),
        in_specs=[
            pl.BlockSpec((bn, din), lambda i: (i, 0)),
            pl.BlockSpec(wt.shape, lambda i: (0, 0)),
            pl.BlockSpec((1, 256), lambda i: (0, 0)),
        ],
        out_specs=[
            pl.BlockSpec((bn, 128), lambda i: (i, 0)),
            pl.BlockSpec((bn, 128), lambda i: (i, 0)),
            pl.BlockSpec((bn, 256), lambda i: (i, 0)),
        ],
        out_shape=[
            jax.ShapeDtypeStruct((n, 128), jnp.float32),
            jax.ShapeDtypeStruct((n, 128), jnp.float32),
            jax.ShapeDtypeStruct((n, 256), jnp.float32),
        ],
    )(x, wt, b)


def _proj_l2(a0, a1, s, wt, b, bn):
    n = s.shape[0]

    def body(a0_ref, a1_ref, s_ref, wt_ref, b_ref, y_ref, s2_ref):
        h0 = jnp.maximum(a0_ref[...] + s_ref[:, :128], 0.0)
        h1 = jnp.maximum(a1_ref[...] + s_ref[:, 128:], 0.0)
        z = (jnp.dot(h0, wt_ref[:128], preferred_element_type=jnp.float32)
             + jnp.dot(h1, wt_ref[128:], preferred_element_type=jnp.float32))
        y_ref[...] = z[:, :128]
        s2_ref[...] = z[:, 128:] + b_ref[...]

    return pl.pallas_call(
        body,
        grid=(n // bn,),
        in_specs=[
            pl.BlockSpec((bn, 128), lambda i: (i, 0)),
            pl.BlockSpec((bn, 128), lambda i: (i, 0)),
            pl.BlockSpec((bn, 256), lambda i: (i, 0)),
            pl.BlockSpec(wt.shape, lambda i: (0, 0)),
            pl.BlockSpec((1, 128), lambda i: (0, 0)),
        ],
        out_specs=[
            pl.BlockSpec((bn, 128), lambda i: (i, 0)),
            pl.BlockSpec((bn, 128), lambda i: (i, 0)),
        ],
        out_shape=[
            jax.ShapeDtypeStruct((n, 128), jnp.float32),
            jax.ShapeDtypeStruct((n, 128), jnp.float32),
        ],
    )(a0, a1, s, wt, b)


def _proj_l3(a2, s, wt, b, bn):
    n = s.shape[0]

    def body(a_ref, s_ref, wt_ref, b_ref, y_ref, s3_ref):
        h = jnp.maximum(a_ref[...] + s_ref[...], 0.0)
        z = jnp.dot(h, wt_ref[...], preferred_element_type=jnp.float32)
        y_ref[...] = jnp.concatenate([z[:, :64], jnp.zeros_like(z[:, :64])], axis=1)
        s3_ref[...] = z[:, 64:] + b_ref[...]

    return pl.pallas_call(
        body,
        grid=(n // bn,),
        in_specs=[
            pl.BlockSpec((bn, 128), lambda i: (i, 0)),
            pl.BlockSpec((bn, 128), lambda i: (i, 0)),
            pl.BlockSpec(wt.shape, lambda i: (0, 0)),
            pl.BlockSpec((1, 64), lambda i: (0, 0)),
        ],
        out_specs=[
            pl.BlockSpec((bn, 128), lambda i: (i, 0)),
            pl.BlockSpec((bn, 64), lambda i: (i, 0)),
        ],
        out_shape=[
            jax.ShapeDtypeStruct((n, 128), jnp.float32),
            jax.ShapeDtypeStruct((n, 64), jnp.float32),
        ],
    )(a2, s, wt, b)


def _proj_head(a3, s, wt, b, bn):
    n = s.shape[0]

    def body(a_ref, s_ref, wt_ref, b_ref, o_ref):
        h = jnp.maximum(a_ref[:, :64] + s_ref[...], 0.0)
        z = jnp.dot(h, wt_ref[...], preferred_element_type=jnp.float32) + b_ref[...]
        col = lax.broadcasted_iota(jnp.int32, z.shape, 1)
        m = (col >= 3) & (col < 7)
        nrm2 = jnp.sum(jnp.where(m, z * z, 0.0), axis=1, keepdims=True)
        denom = jnp.maximum(jnp.sqrt(nrm2), 1e-12)
        o_ref[...] = jnp.where(m, z / denom, z)

    return pl.pallas_call(
        body,
        grid=(n // bn,),
        in_specs=[
            pl.BlockSpec((bn, 128), lambda i: (i, 0)),
            pl.BlockSpec((bn, 64), lambda i: (i, 0)),
            pl.BlockSpec(wt.shape, lambda i: (0, 0)),
            pl.BlockSpec((1, 8), lambda i: (0, 0)),
        ],
        out_specs=pl.BlockSpec((bn, 8), lambda i: (i, 0)),
        out_shape=jax.ShapeDtypeStruct((n, 8), jnp.float32),
    )(a3, s, wt, b)


def kernel(x, edge_index, W1r, W1s, b1, W2r, W2s, b2, W3r, W3s, b3, Wp, bp, Wo, bo):
    src = edge_index[0]
    dst = edge_index[1]
    pad = _EPAD - _E
    srcr = jnp.concatenate(
        [src, jnp.zeros((pad,), jnp.int32)]).reshape(_NSUB, _CPS, _CHUNK)
    dstp = jnp.concatenate([dst, jnp.full((pad,), _N, jnp.int32)])
    pos = jnp.arange(_EPAD, dtype=jnp.int32)
    dum0 = _HALF + (pos % (_ACCR - _HALF))
    dum1 = (_N - _HALF + 1) + (pos % (_DUMMY - (_N - _HALF)))
    loc0 = jnp.where(dstp < _HALF, dstp, dum0).reshape(_NSUB, _CPS, _CHUNK)
    loc1 = jnp.where(dstp >= _HALF, jnp.minimum(dstp - _HALF, _DUMMY),
                     dum1).reshape(_NSUB, _CPS, _CHUNK)
    dstn = jnp.stack([loc0, loc1])


    wt1 = jnp.concatenate([W1r, W1s], axis=0).T
    wt2 = jnp.concatenate([W2r, W2s], axis=0).T
    wt3 = jnp.concatenate([W3r, W3s], axis=0).T
    wth = jnp.concatenate(
        [Wp, Wo, jnp.zeros((1, 64), jnp.float32)], axis=0).T
    bh = jnp.concatenate([bp, bo, jnp.zeros((1,), jnp.float32)]).reshape(1, 8)
    zer = jnp.zeros((_CHUNK, 128), jnp.float32)
    sc1 = _seg_scatter(1)
    sc2 = _seg_scatter(2)

    y1a, y1b, s1 = _proj_first(x, wt1, b1.reshape(1, -1), 1000)
    a0, a1 = sc2(y1a, y1b, srcr, dstn, zer)
    y2, s2 = _proj_l2(a0, a1, s1, wt2, b2.reshape(1, -1), 1000)
    a2, = sc1(y2, srcr, dstn, zer)
    y3, s3 = _proj_l3(a2, s2, wt3, b3.reshape(1, -1), 1000)
    a3, = sc1(y3, srcr, dstn, zer)
    out8 = _proj_head(a3, s3, wth, bh, 1000)
    return (out8[:, 0:3], out8[:, 3:7])


# speedup vs baseline: 1.4398x; 1.0003x over previous
import functools

import jax
import jax.numpy as jnp
from jax import lax
from jax.experimental import pallas as pl
from jax.experimental.pallas import tpu as pltpu
from jax.experimental.pallas import tpu_sc as plsc

_N = 10000
_E = 40000
_NSUB = 16
_CHUNK = 128
_CPS = 20
_EPAD = _NSUB * _CPS * _CHUNK
_HALF = 4992
_ACCR = 5120
_RPS = 320
_C0R = 312
_OUTR = _HALF + _ACCR
_DUMMY = _ACCR - 1


def _seg_scatter(ntab):
    mesh = plsc.VectorSubcoreMesh(core_axis_name="c", subcore_axis_name="s")

    @functools.partial(
        pl.kernel,
        out_type=[jax.ShapeDtypeStruct((_OUTR, 128), jnp.float32)] * ntab,
        mesh=mesh,
        scratch_types=[
            pltpu.VMEM((_CPS, _CHUNK), jnp.int32),
            pltpu.VMEM((_CPS, _CHUNK), jnp.int32),
            pltpu.VMEM((3, _CHUNK, 128), jnp.float32),
            pltpu.VMEM_SHARED((_ACCR, 128), jnp.float32),
            pltpu.SemaphoreType.DMA,
            pltpu.SemaphoreType.DMA,
        ],
        compiler_params=pltpu.CompilerParams(needs_layout_passes=False),
    )
    def scatter_kernel(*refs):
        ytabs = refs[:ntab]
        srcr, dstn, zeros = refs[ntab:ntab + 3]
        outs = refs[ntab + 3:2 * ntab + 3]
        src_v, dst_v, rows_v, acc, sem, sem2 = refs[2 * ntab + 3:]
        c = lax.axis_index("c")
        s = lax.axis_index("s")
        pltpu.sync_copy(srcr.at[s], src_v)
        pltpu.sync_copy(dstn.at[c, s], dst_v)

        for t in range(ntab):
            ytab = ytabs[t]
            out = outs[t]
            pltpu.sync_copy(zeros, rows_v.at[0])
            base = s * _RPS
            pltpu.sync_copy(rows_v.at[0], acc.at[pl.ds(base, _CHUNK)])
            pltpu.sync_copy(rows_v.at[0], acc.at[pl.ds(base + _CHUNK, _CHUNK)])
            pltpu.sync_copy(rows_v.at[0].at[pl.ds(0, _RPS - 2 * _CHUNK)],
                            acc.at[pl.ds(base + 2 * _CHUNK, _RPS - 2 * _CHUNK)])
            plsc.subcore_barrier()

            def chunk(j, carry, ytab=ytab):
                pltpu.make_async_copy(ytab.at[src_v.at[j]], rows_v.at[j % 3],
                                      sem).wait()
                pltpu.async_copy(rows_v.at[j % 3], acc.at[dst_v.at[j]], sem2,
                                 add=True)

                @pl.when(j >= 1)
                def _():
                    pltpu.make_async_copy(ytab.at[src_v.at[0]], rows_v.at[0],
                                          sem2).wait()

                @pl.when(j + 2 < _CPS)
                def _():
                    pltpu.async_copy(ytab.at[src_v.at[j + 2]],
                                     rows_v.at[(j + 2) % 3], sem)

                return carry

            for b in range(2):
                pltpu.async_copy(ytab.at[src_v.at[b]], rows_v.at[b], sem)
            lax.fori_loop(0, _CPS, chunk, 0)
            pltpu.make_async_copy(ytab.at[src_v.at[0]], rows_v.at[0],
                                  sem2).wait()
            plsc.subcore_barrier()

            @pl.when(c == 0)
            def _():
                for p, (off, ln) in enumerate(((0, _CHUNK), (_CHUNK, _CHUNK),
                                               (2 * _CHUNK, _C0R - 2 * _CHUNK))):
                    piece = rows_v.at[p].at[pl.ds(0, ln)]
                    pltpu.sync_copy(acc.at[pl.ds(s * _C0R + off, ln)], piece)
                    pltpu.sync_copy(piece, out.at[pl.ds(s * _C0R + off, ln)])

            @pl.when(c == 1)
            def _():
                for p, (off, ln) in enumerate(((0, _CHUNK), (_CHUNK, _CHUNK),
                                               (2 * _CHUNK, _RPS - 2 * _CHUNK))):
                    piece = rows_v.at[p].at[pl.ds(0, ln)]
                    pltpu.sync_copy(acc.at[pl.ds(s * _RPS + off, ln)], piece)
                    pltpu.sync_copy(piece,
                                    out.at[pl.ds(_HALF + s * _RPS + off, ln)])

            if t + 1 < ntab:
                plsc.subcore_barrier()

    return scatter_kernel


def _proj_first(x, wt, b, bn):
    n, din = x.shape

    def body(x_ref, wt_ref, b_ref, ya_ref, yb_ref, s_ref):
        z = jnp.dot(x_ref[...], wt_ref[...], preferred_element_type=jnp.float32)
        ya_ref[...] = z[:, :128]
        yb_ref[...] = z[:, 128:256]
        s_ref[...] = z[:, 256:] + b_ref[...]

    return pl.pallas_call(
        body,
        grid=(n // bn,),
        in_specs=[
            pl.BlockSpec((bn, din), lambda i: (i, 0)),
            pl.BlockSpec(wt.shape, lambda i: (0, 0)),
            pl.BlockSpec((1, 256), lambda i: (0, 0)),
        ],
        out_specs=[
            pl.BlockSpec((bn, 128), lambda i: (i, 0)),
            pl.BlockSpec((bn, 128), lambda i: (i, 0)),
            pl.BlockSpec((bn, 256), lambda i: (i, 0)),
        ],
        out_shape=[
            jax.ShapeDtypeStruct((n, 128), jnp.float32),
            jax.ShapeDtypeStruct((n, 128), jnp.float32),
            jax.ShapeDtypeStruct((n, 256), jnp.float32),
        ],
    )(x, wt, b)


def _proj_l2(a0, a1, s, wt, b, bn):
    n = s.shape[0]

    def body(a0_ref, a1_ref, s_ref, wt_ref, b_ref, y_ref, s2_ref):
        h0 = jnp.maximum(a0_ref[...] + s_ref[:, :128], 0.0)
        h1 = jnp.maximum(a1_ref[...] + s_ref[:, 128:], 0.0)
        z = (jnp.dot(h0, wt_ref[:128], preferred_element_type=jnp.float32)
             + jnp.dot(h1, wt_ref[128:], preferred_element_type=jnp.float32))
        y_ref[...] = z[:, :128]
        s2_ref[...] = z[:, 128:] + b_ref[...]

    return pl.pallas_call(
        body,
        grid=(n // bn,),
        in_specs=[
            pl.BlockSpec((bn, 128), lambda i: (i, 0)),
            pl.BlockSpec((bn, 128), lambda i: (i, 0)),
            pl.BlockSpec((bn, 256), lambda i: (i, 0)),
            pl.BlockSpec(wt.shape, lambda i: (0, 0)),
            pl.BlockSpec((1, 128), lambda i: (0, 0)),
        ],
        out_specs=[
            pl.BlockSpec((bn, 128), lambda i: (i, 0)),
            pl.BlockSpec((bn, 128), lambda i: (i, 0)),
        ],
        out_shape=[
            jax.ShapeDtypeStruct((n, 128), jnp.float32),
            jax.ShapeDtypeStruct((n, 128), jnp.float32),
        ],
    )(a0, a1, s, wt, b)


def _proj_l3(a2, s, wt, b, bn):
    n = s.shape[0]

    def body(a_ref, s_ref, wt_ref, b_ref, y_ref, s3_ref):
        h = jnp.maximum(a_ref[...] + s_ref[...], 0.0)
        z = jnp.dot(h, wt_ref[...], preferred_element_type=jnp.float32)
        y_ref[...] = jnp.concatenate([z[:, :64], jnp.zeros_like(z[:, :64])], axis=1)
        s3_ref[...] = z[:, 64:] + b_ref[...]

    return pl.pallas_call(
        body,
        grid=(n // bn,),
        in_specs=[
            pl.BlockSpec((bn, 128), lambda i: (i, 0)),
            pl.BlockSpec((bn, 128), lambda i: (i, 0)),
            pl.BlockSpec(wt.shape, lambda i: (0, 0)),
            pl.BlockSpec((1, 64), lambda i: (0, 0)),
        ],
        out_specs=[
            pl.BlockSpec((bn, 128), lambda i: (i, 0)),
            pl.BlockSpec((bn, 64), lambda i: (i, 0)),
        ],
        out_shape=[
            jax.ShapeDtypeStruct((n, 128), jnp.float32),
            jax.ShapeDtypeStruct((n, 64), jnp.float32),
        ],
    )(a2, s, wt, b)


def _proj_head(a3, s, wt, b, bn):
    n = s.shape[0]

    def body(a_ref, s_ref, wt_ref, b_ref, o_ref):
        h = jnp.maximum(a_ref[:, :64] + s_ref[...], 0.0)
        z = jnp.dot(h, wt_ref[...], preferred_element_type=jnp.float32) + b_ref[...]
        col = lax.broadcasted_iota(jnp.int32, z.shape, 1)
        m = (col >= 3) & (col < 7)
        nrm2 = jnp.sum(jnp.where(m, z * z, 0.0), axis=1, keepdims=True)
        denom = jnp.maximum(jnp.sqrt(nrm2), 1e-12)
        o_ref[...] = jnp.where(m, z / denom, z)

    return pl.pallas_call(
        body,
        grid=(n // bn,),
        in_specs=[
            pl.BlockSpec((bn, 128), lambda i: (i, 0)),
            pl.BlockSpec((bn, 64), lambda i: (i, 0)),
            pl.BlockSpec(wt.shape, lambda i: (0, 0)),
            pl.BlockSpec((1, 8), lambda i: (0, 0)),
        ],
        out_specs=pl.BlockSpec((bn, 8), lambda i: (i, 0)),
        out_shape=jax.ShapeDtypeStruct((n, 8), jnp.float32),
    )(a3, s, wt, b)


def kernel(x, edge_index, W1r, W1s, b1, W2r, W2s, b2, W3r, W3s, b3, Wp, bp, Wo, bo):
    src = edge_index[0]
    dst = edge_index[1]
    pad = _EPAD - _E
    srcr = jnp.concatenate(
        [src, jnp.zeros((pad,), jnp.int32)]).reshape(_NSUB, _CPS, _CHUNK)
    dstp = jnp.concatenate([dst, jnp.full((pad,), _N, jnp.int32)])
    pos = jnp.arange(_EPAD, dtype=jnp.int32)
    dum0 = _HALF + (pos % (_ACCR - _HALF))
    dum1 = (_N - _HALF + 1) + (pos % (_DUMMY - (_N - _HALF)))
    loc0 = jnp.where(dstp < _HALF, dstp, dum0).reshape(_NSUB, _CPS, _CHUNK)
    loc1 = jnp.where(dstp >= _HALF, jnp.minimum(dstp - _HALF, _DUMMY),
                     dum1).reshape(_NSUB, _CPS, _CHUNK)
    dstn = jnp.stack([loc0, loc1])


    wt1 = jnp.concatenate([W1r, W1s], axis=0).T
    wt2 = jnp.concatenate([W2r, W2s], axis=0).T
    wt3 = jnp.concatenate([W3r, W3s], axis=0).T
    wth = jnp.concatenate(
        [Wp, Wo, jnp.zeros((1, 64), jnp.float32)], axis=0).T
    bh = jnp.concatenate([bp, bo, jnp.zeros((1,), jnp.float32)]).reshape(1, 8)
    zer = jnp.zeros((_CHUNK, 128), jnp.float32)
    sc1 = _seg_scatter(1)
    sc2 = _seg_scatter(2)

    y1a, y1b, s1 = _proj_first(x, wt1, b1.reshape(1, -1), 1000)
    a0, a1 = sc2(y1a, y1b, srcr, dstn, zer)
    y2, s2 = _proj_l2(a0, a1, s1, wt2, b2.reshape(1, -1), 1000)
    a2, = sc1(y2, srcr, dstn, zer)
    y3, s3 = _proj_l3(a2, s2, wt3, b3.reshape(1, -1), 1000)
    a3, = sc1(y3, srcr, dstn, zer)
    out8 = _proj_head(a3, s3, wth, bh, 1000)
    return (out8[:, 0:3], out8[:, 3:7])
